# Initial kernel scaffold; baseline (speedup 1.0000x reference)
import jax
import jax.numpy as jnp
from jax.experimental import pallas as pl


def kernel(points, features, leaf_mask, W1, b1, W2, b2):
    raise NotImplementedError("write your pallas kernel here")


# speedup vs baseline: 41.8936x; 41.8936x over previous
import functools

import jax
import jax.numpy as jnp
from jax import lax
from jax.experimental import pallas as pl

B, N, D_IN, D_OUT, K = 2, 4096, 128, 256, 10
TM = 256
_INF = 1e30


def _body(ptsT_ref, ptsN_ref, p16_ref, f_ref, w1a_ref, w1b_ref, b1_ref,
          w2_ref, b2_ref, o_ref):
    t = pl.program_id(1)
    PT = ptsT_ref[0]
    Xr = PT[0:1, :]
    Yr = PT[1:2, :]
    Zr = PT[2:3, :]
    Mr = PT[3:4, :]

    msum = jnp.sum(Mr, keepdims=True)
    denom = jnp.maximum(msum, 1.0)
    cx = jnp.sum(Xr * Mr, keepdims=True) / denom
    cy = jnp.sum(Yr * Mr, keepdims=True) / denom
    cz = jnp.sum(Zr * Mr, keepdims=True) / denom

    tile = ptsN_ref[0, pl.ds(t * TM, TM), :]
    xi = tile[:, 0:1]
    yi = tile[:, 1:2]
    zi = tile[:, 2:3]

    dx = xi - Xr
    dy = yi - Yr
    dz = zi - Zr
    d2 = (dx * dx + dy * dy) + dz * dz

    density = jnp.sum(
        jnp.where((d2 < (0.02 ** 2)) & (Mr > 0.0), 1.0, 0.0),
        axis=1, keepdims=True)

    iota = lax.broadcasted_iota(jnp.int32, (1, N), 1)
    work = d2
    S = jnp.zeros((TM, N), jnp.float32)
    for _ in range(K):
        m = jnp.min(work, axis=1, keepdims=True)
        eq = work == m
        ji = jnp.min(jnp.where(eq, iota, N), axis=1, keepdims=True)
        sel = iota == ji
        S = S + sel.astype(jnp.float32)
        work = jnp.where(sel, _INF, work)

    M = jnp.dot(S, p16_ref[0], preferred_element_type=jnp.float32,
                precision=jax.lax.Precision.HIGHEST)
    m1x = M[:, 0:1]
    m1y = M[:, 1:2]
    m1z = M[:, 2:3]
    Kf = float(K)
    cxx = (M[:, 3:4] - 2.0 * xi * m1x + Kf * xi * xi) / Kf
    cyy = (M[:, 4:5] - 2.0 * yi * m1y + Kf * yi * yi) / Kf
    czz = (M[:, 5:6] - 2.0 * zi * m1z + Kf * zi * zi) / Kf
    cxy = (M[:, 6:7] - xi * m1y - yi * m1x + Kf * xi * yi) / Kf
    cxz = (M[:, 7:8] - xi * m1z - zi * m1x + Kf * xi * zi) / Kf
    cyz = (M[:, 8:9] - yi * m1z - zi * m1y + Kf * yi * zi) / Kf

    q = (cxx + cyy + czz) / 3.0
    p1 = cxy * cxy + cxz * cxz + cyz * cyz
    dxx = cxx - q
    dyy = cyy - q
    dzz = czz - q
    p2 = dxx * dxx + dyy * dyy + dzz * dzz + 2.0 * p1
    degen = p2 <= 1e-22
    p = jnp.sqrt(jnp.maximum(p2, 1e-22) / 6.0)
    bxx = dxx / p
    byy = dyy / p
    bzz = dzz / p
    bxy = cxy / p
    bxz = cxz / p
    byz = cyz / p
    detb = (bxx * (byy * bzz - byz * byz)
            - bxy * (bxy * bzz - byz * bxz)
            + bxz * (bxy * byz - byy * bxz))
    r = jnp.clip(detb * 0.5, -1.0, 1.0)
    phi = jnp.arctan2(jnp.sqrt(jnp.maximum(1.0 - r * r, 0.0)), r) / 3.0
    ev2 = q + 2.0 * p * jnp.cos(phi)
    ev0 = q + 2.0 * p * jnp.cos(phi + 2.0 * jnp.pi / 3.0)
    ev2 = jnp.where(degen, q, ev2)
    ev0 = jnp.where(degen, q, ev0)
    curv = ev0 / (ev2 + 1e-8)

    dxc = xi - cx
    dyc = yi - cy
    dzc = zi - cz
    dist_c = jnp.sqrt(dxc * dxc + dyc * dyc + dzc * dzc)
    hdist = jnp.sqrt(dxc * dxc + dyc * dyc)
    rad = jnp.arctan2(dyc, dxc)

    gate = (msum > 0.0).astype(jnp.float32)
    w1b = w1b_ref[...]
    gcon = (dist_c * w1b[0:1, :] + dzc * w1b[1:2, :]
            + hdist * w1b[2:3, :] + density * w1b[3:4, :]
            + curv * w1b[4:5, :] + rad * w1b[5:6, :]) * gate

    h = jnp.dot(f_ref[0], w1a_ref[...], preferred_element_type=jnp.float32)
    h = jnp.maximum(h + gcon + b1_ref[...], 0.0)
    out = jnp.dot(h, w2_ref[...], preferred_element_type=jnp.float32)
    o_ref[0] = jnp.maximum(out + b2_ref[...], 0.0)


@jax.jit
def kernel(points, features, leaf_mask, W1, b1, W2, b2):
    maskf = leaf_mask.astype(jnp.float32)
    ptsT = jnp.concatenate(
        [points.transpose(0, 2, 1), maskf[:, None, :],
         jnp.zeros((B, 4, N), jnp.float32)], axis=1)
    ptsN = jnp.pad(points, ((0, 0), (0, 0), (0, 5)))
    x, y, z = points[..., 0:1], points[..., 1:2], points[..., 2:3]
    p16 = jnp.concatenate(
        [x, y, z, x * x, y * y, z * z, x * y, x * z, y * z,
         jnp.zeros((B, N, 7), jnp.float32)], axis=-1)
    w1a = W1[:D_IN, :]
    w1b = jnp.pad(W1[D_IN:, :], ((0, 2), (0, 0)))
    b1r = b1[None, :]
    b2r = b2[None, :]

    grid = (B, N // TM)
    return pl.pallas_call(
        _body,
        grid=grid,
        in_specs=[
            pl.BlockSpec((1, 8, N), lambda b, t: (b, 0, 0)),
            pl.BlockSpec((1, N, 8), lambda b, t: (b, 0, 0)),
            pl.BlockSpec((1, N, 16), lambda b, t: (b, 0, 0)),
            pl.BlockSpec((1, TM, D_IN), lambda b, t: (b, t, 0)),
            pl.BlockSpec((D_IN, D_OUT), lambda b, t: (0, 0)),
            pl.BlockSpec((8, D_OUT), lambda b, t: (0, 0)),
            pl.BlockSpec((1, D_OUT), lambda b, t: (0, 0)),
            pl.BlockSpec((D_OUT, D_OUT), lambda b, t: (0, 0)),
            pl.BlockSpec((1, D_OUT), lambda b, t: (0, 0)),
        ],
        out_specs=pl.BlockSpec((1, TM, D_OUT), lambda b, t: (b, t, 0)),
        out_shape=jax.ShapeDtypeStruct((B, N, D_OUT), jnp.float32),
    )(ptsT, ptsN, p16, features, w1a, w1b, b1r, W2, b2r)
